# Initial kernel scaffold; baseline (speedup 1.0000x reference)
import jax
import jax.numpy as jnp
from jax.experimental import pallas as pl


def kernel(x, W1, g1, b1, W2, g2, b2, W3, g3, b3, W4, g4, b4, Wc1, bc1, gc1, bnc1, Wc2, bc2, gc2, bnc2, Wc3, bc3):
    raise NotImplementedError("write your pallas kernel here")


# speedup vs baseline: 10.9501x; 10.9501x over previous
import functools

import jax
import jax.numpy as jnp
from jax import lax
from jax.experimental import pallas as pl
from jax.experimental.pallas import tpu as pltpu
from jax.experimental.pallas import tpu_sc as plsc

KNN = 20
BLK = 256
NC, NS = 2, 16
NW = NC * NS


def _sq_kernel(xt_ref, xc_ref, sqr_ref, sqc_ref, *, lane_path):
    if lane_path:
        sc = jnp.sum(xt_ref[0] * xt_ref[0], axis=1, keepdims=True)
        sqc_ref[0] = sc
        sqr_ref[0] = lax.transpose(sc, (1, 0))
    else:
        sr = jnp.sum(xc_ref[0] * xc_ref[0], axis=0, keepdims=True)
        sqr_ref[0] = sr
        sqc_ref[0] = lax.transpose(sr, (1, 0))


def _knn_kernel(xb_ref, xc_ref, sqr_ref, sqc_ref, idx_ref):
    b = pl.program_id(0)
    xb = xb_ref[0]
    xc = xc_ref[0]
    n = xc.shape[1]
    sq_j = sqr_ref[0]
    sq_i = sqc_ref[0]
    inner = -2.0 * lax.dot_general(
        xb, xc, (((1,), (0,)), ((), ())),
        preferred_element_type=jnp.float32)
    score = (-sq_i - inner) - sq_j

    col = lax.broadcasted_iota(jnp.int32, (xb.shape[0], n), 1)
    lanek = lax.broadcasted_iota(jnp.int32, (xb.shape[0], KNN), 1)
    acc = jnp.zeros((xb.shape[0], KNN), jnp.int32)
    neginf = jnp.float32(-jnp.inf)
    for t in range(KNN):
        mv = jnp.max(score, axis=1, keepdims=True)
        am = jnp.min(jnp.where(score == mv, col, n), axis=1).astype(jnp.int32)
        acc = jnp.where(lanek == t, am[:, None] + b * n, acc)
        if t != KNN - 1:
            score = jnp.where(col == am[:, None], neginf, score)
    idx_ref[0] = acc


def _layer_knn(xT):
    B, N, C = xT.shape
    xC = jnp.transpose(xT, (0, 2, 1))
    sqr, sqc = pl.pallas_call(
        functools.partial(_sq_kernel, lane_path=(C < 8)),
        grid=(B,),
        in_specs=[
            pl.BlockSpec((1, N, C), lambda b: (b, 0, 0)),
            pl.BlockSpec((1, C, N), lambda b: (b, 0, 0)),
        ],
        out_specs=[
            pl.BlockSpec((1, 1, N), lambda b: (b, 0, 0)),
            pl.BlockSpec((1, N, 1), lambda b: (b, 0, 0)),
        ],
        out_shape=[
            jax.ShapeDtypeStruct((B, 1, N), jnp.float32),
            jax.ShapeDtypeStruct((B, N, 1), jnp.float32),
        ],
    )(xT, xC)
    return pl.pallas_call(
        _knn_kernel,
        grid=(B, N // BLK),
        in_specs=[
            pl.BlockSpec((1, BLK, C), lambda b, i: (b, i, 0)),
            pl.BlockSpec((1, C, N), lambda b, i: (b, 0, 0)),
            pl.BlockSpec((1, 1, N), lambda b, i: (b, 0, 0)),
            pl.BlockSpec((1, BLK, 1), lambda b, i: (b, i, 0)),
        ],
        out_specs=pl.BlockSpec((1, BLK, KNN), lambda b, i: (b, i, 0)),
        out_shape=jax.ShapeDtypeStruct((B, N, KNN), jnp.int32),
    )(xT, xC, sqr, sqc)


def _sc_gather_body(table_hbm, idx_hbm, g_hbm, idx_v, rows_v, sem, *, PPW, GP, NG):
    wid = lax.axis_index("s") * NC + lax.axis_index("c")
    base = wid * PPW

    def group(g, carry):
        rbase = (base + g * GP) * KNN
        pltpu.sync_copy(idx_hbm.at[pl.ds(rbase, GP * KNN)], idx_v)
        pltpu.async_copy(table_hbm.at[idx_v], rows_v, sem).wait()
        pltpu.sync_copy(rows_v, g_hbm.at[pl.ds(rbase, GP * KNN)])
        return carry

    lax.fori_loop(0, NG, group, 0)


def _sc_gather(table, idx_flat):
    M, Cp = table.shape
    R = idx_flat.shape[0]
    PPW = (R // KNN) // NW
    GP = 8
    NG = PPW // GP
    mesh = plsc.VectorSubcoreMesh(core_axis_name="c", subcore_axis_name="s")
    kcall = pl.kernel(
        functools.partial(_sc_gather_body, PPW=PPW, GP=GP, NG=NG),
        out_type=jax.ShapeDtypeStruct((R, Cp), jnp.float32),
        mesh=mesh,
        scratch_types=[
            pltpu.VMEM((GP * KNN,), jnp.int32),
            pltpu.VMEM((GP * KNN, Cp), jnp.float32),
            pltpu.SemaphoreType.DMA,
        ],
    )
    return kcall(table, idx_flat)


def _edge_kernel(g_ref, xb_ref, wt_ref, m_ref, ps_ref, ps2_ref):
    C2, Op = wt_ref.shape
    C = C2 // 2
    G = g_ref[...]
    xr = xb_ref[0]
    Gc = G[:, :C]
    xrc = xr[:, :C]
    xrep = jnp.broadcast_to(
        xrc[:, None, :], (xrc.shape[0], KNN, C)).reshape(xrc.shape[0] * KNN, C)
    f = jnp.concatenate([Gc - xrep, xrep], axis=1)
    y = lax.dot_general(
        f, wt_ref[...], (((1,), (0,)), ((), ())),
        preferred_element_type=jnp.float32)
    m_ref[0] = jnp.max(y.reshape(xrc.shape[0], KNN, Op), axis=1)
    ps_ref[0] = jnp.sum(y, axis=0, keepdims=True)
    ps2_ref[0] = jnp.sum(y * y, axis=0, keepdims=True)


def _layer_edge(G, xTp, WT):
    B, N, Cp = xTp.shape
    NB = N // BLK
    C2, Op = WT.shape
    nblk = B * NB
    return pl.pallas_call(
        _edge_kernel,
        grid=(B, NB),
        in_specs=[
            pl.BlockSpec((BLK * KNN, Cp), lambda b, i: (b * (N // BLK) + i, 0)),
            pl.BlockSpec((1, BLK, Cp), lambda b, i: (b, i, 0)),
            pl.BlockSpec((C2, Op), lambda b, i: (0, 0)),
        ],
        out_specs=[
            pl.BlockSpec((1, BLK, Op), lambda b, i: (b, i, 0)),
            pl.BlockSpec((1, 1, Op), lambda b, i: (b * (N // BLK) + i, 0, 0)),
            pl.BlockSpec((1, 1, Op), lambda b, i: (b * (N // BLK) + i, 0, 0)),
        ],
        out_shape=[
            jax.ShapeDtypeStruct((B, N, Op), jnp.float32),
            jax.ShapeDtypeStruct((nblk, 1, Op), jnp.float32),
            jax.ShapeDtypeStruct((nblk, 1, Op), jnp.float32),
        ],
    )(G, xTp, WT)


def _combine_kernel(m_ref, ps_ref, ps2_ref, g_ref, beta_ref, o_ref, *, cnt):
    mean = jnp.sum(ps_ref[...], axis=0, keepdims=True) / cnt
    ey2 = jnp.sum(ps2_ref[...], axis=0, keepdims=True) / cnt
    var = ey2 - mean * mean
    y = (m_ref[...] - mean) / jnp.sqrt(var + 1e-5) * g_ref[...] + beta_ref[...]
    o_ref[...] = jnp.where(y > 0, y, 0.2 * y)


def _combine(M2, ps, ps2, g, beta, cnt):
    Mrows, Op = M2.shape
    return pl.pallas_call(
        functools.partial(_combine_kernel, cnt=cnt),
        out_shape=jax.ShapeDtypeStruct((Mrows, Op), jnp.float32),
    )(M2, ps, ps2, g.reshape(1, Op), beta.reshape(1, Op))


_HBLK = 1024


def _mm_bias_kernel(x_ref, w_ref, b_ref, o_ref):
    o_ref[...] = b_ref[...] + jnp.dot(x_ref[...], w_ref[...],
                                      preferred_element_type=jnp.float32)


def _mm_bias(x, w, bias):
    M, _ = x.shape
    O = w.shape[1]
    return pl.pallas_call(
        _mm_bias_kernel,
        grid=(M // _HBLK,),
        in_specs=[pl.BlockSpec((_HBLK, x.shape[1]), lambda i: (i, 0)),
                  pl.BlockSpec(w.shape, lambda i: (0, 0)),
                  pl.BlockSpec((1, O), lambda i: (0, 0))],
        out_specs=pl.BlockSpec((_HBLK, O), lambda i: (i, 0)),
        out_shape=jax.ShapeDtypeStruct((M, O), jnp.float32),
    )(x, w, bias.reshape(1, O))


def _bn_relu_kernel(h_ref, g_ref, beta_ref, o_ref):
    h = h_ref[...]
    mean = jnp.mean(h, axis=0, keepdims=True)
    var = jnp.mean((h - mean) ** 2, axis=0, keepdims=True)
    o_ref[...] = jnp.maximum(
        (h - mean) / jnp.sqrt(var + 1e-5) * g_ref[...] + beta_ref[...], 0.0)


def _bn_relu(h, g, beta):
    M, O = h.shape
    return pl.pallas_call(
        _bn_relu_kernel,
        out_shape=jax.ShapeDtypeStruct((M, O), jnp.float32),
    )(h, g.reshape(1, O), beta.reshape(1, O))


def _edge_layer(xT_real, W, g, beta):
    B, N, C = xT_real.shape
    O = W.shape[0]
    Op = -(-O // 128) * 128
    Cp = -(-C // 128) * 128
    idx = _layer_knn(xT_real)
    xTp = xT_real if Cp == C else jnp.pad(xT_real, ((0, 0), (0, 0), (0, Cp - C)))
    G = _sc_gather(xTp.reshape(B * N, Cp), idx.reshape(-1))
    WT = jnp.zeros((2 * C, Op), jnp.float32).at[:, :O].set(W.T)
    gp = jnp.zeros((Op,), jnp.float32).at[:O].set(g)
    betap = jnp.zeros((Op,), jnp.float32).at[:O].set(beta)
    M2, ps, ps2 = _layer_edge(G, xTp, WT)
    out = _combine(M2.reshape(B * N, Op), ps.reshape(-1, Op),
                   ps2.reshape(-1, Op), gp, betap, float(B * N * KNN))
    return out.reshape(B, N, Op)


def kernel(x, W1, g1, b1, W2, g2, b2, W3, g3, b3, W4, g4, b4,
           Wc1, bc1, gc1, bnc1, Wc2, bc2, gc2, bnc2, Wc3, bc3):
    B, N, _ = x.shape
    sizes = (64, 64, 128, 256)
    xT = x
    reals = []
    for (W, g, beta), O in zip(((W1, g1, b1), (W2, g2, b2),
                                (W3, g3, b3), (W4, g4, b4)), sizes):
        xTp = _edge_layer(xT, W, g, beta)
        xT = xTp[..., :O]
        reals.append(xT.reshape(B * N, O))
    xcat = jnp.concatenate(reals, axis=1)
    h = _mm_bias(xcat, Wc1.T, bc1)
    h = _bn_relu(h, gc1, bnc1)
    h = _mm_bias(h, Wc2.T, bc2)
    h = _bn_relu(h, gc2, bnc2)
    out = _mm_bias(h, Wc3.T, bc3)
    return out.reshape(B, N)


# speedup vs baseline: 12.2224x; 1.1162x over previous
import functools

import jax
import jax.numpy as jnp
from jax import lax
from jax.experimental import pallas as pl
from jax.experimental.pallas import tpu as pltpu
from jax.experimental.pallas import tpu_sc as plsc

KNN = 20
BLK = 256
NC, NS = 2, 16
NW = NC * NS


def _sq_kernel(xt_ref, xc_ref, sqr_ref, sqc_ref, *, lane_path):
    if lane_path:
        sc = jnp.sum(xt_ref[0] * xt_ref[0], axis=1, keepdims=True)
        sqc_ref[0] = sc
        sqr_ref[0] = lax.transpose(sc, (1, 0))
    else:
        sr = jnp.sum(xc_ref[0] * xc_ref[0], axis=0, keepdims=True)
        sqr_ref[0] = sr
        sqc_ref[0] = lax.transpose(sr, (1, 0))


def _knn_kernel(xb_ref, xcr_ref, sqrr_ref, sqc_ref, idx_ref):
    b = pl.program_id(0)
    xb = xb_ref[0]
    xcr = xcr_ref[0]
    n = xcr.shape[1]
    sq_jr = sqrr_ref[0]
    sq_i = sqc_ref[0]
    inner = -2.0 * lax.dot_general(
        xb, xcr, (((1,), (0,)), ((), ())),
        preferred_element_type=jnp.float32)
    score = (-sq_i - inner) - sq_jr

    col = lax.broadcasted_iota(jnp.int32, (xb.shape[0], n), 1)
    lanek = lax.broadcasted_iota(jnp.int32, (xb.shape[0], KNN), 1)
    acc = jnp.zeros((xb.shape[0], KNN), jnp.int32)
    neginf = jnp.float32(-jnp.inf)
    base = b * n + (n - 1)
    for t in range(KNN):
        am = jnp.argmax(score, axis=1).astype(jnp.int32)
        acc = jnp.where(lanek == t, base - am[:, None], acc)
        if t != KNN - 1:
            score = jnp.where(col == am[:, None], neginf, score)
    idx_ref[0] = acc


def _layer_knn(xT):
    B, N, C = xT.shape
    xC = jnp.transpose(xT, (0, 2, 1))
    sqr, sqc = pl.pallas_call(
        functools.partial(_sq_kernel, lane_path=(C < 8)),
        grid=(B,),
        in_specs=[
            pl.BlockSpec((1, N, C), lambda b: (b, 0, 0)),
            pl.BlockSpec((1, C, N), lambda b: (b, 0, 0)),
        ],
        out_specs=[
            pl.BlockSpec((1, 1, N), lambda b: (b, 0, 0)),
            pl.BlockSpec((1, N, 1), lambda b: (b, 0, 0)),
        ],
        out_shape=[
            jax.ShapeDtypeStruct((B, 1, N), jnp.float32),
            jax.ShapeDtypeStruct((B, N, 1), jnp.float32),
        ],
    )(xT, xC)
    xCr = jnp.flip(xC, axis=2)
    sqrr = jnp.flip(sqr, axis=2)
    return pl.pallas_call(
        _knn_kernel,
        grid=(B, N // BLK),
        in_specs=[
            pl.BlockSpec((1, BLK, C), lambda b, i: (b, i, 0)),
            pl.BlockSpec((1, C, N), lambda b, i: (b, 0, 0)),
            pl.BlockSpec((1, 1, N), lambda b, i: (b, 0, 0)),
            pl.BlockSpec((1, BLK, 1), lambda b, i: (b, i, 0)),
        ],
        out_specs=pl.BlockSpec((1, BLK, KNN), lambda b, i: (b, i, 0)),
        out_shape=jax.ShapeDtypeStruct((B, N, KNN), jnp.int32),
    )(xT, xCr, sqrr, sqc)


def _sc_gather_body(table_hbm, idx_hbm, g_hbm,
                    idx_v0, idx_v1, rows_v0, rows_v1, sem0, sem1,
                    *, PPW, GP, NG):
    wid = lax.axis_index("s") * NC + lax.axis_index("c")
    base = wid * PPW
    GR = GP * KNN

    def rb(g):
        return (base + g * GP) * KNN

    def fire(g, idx_v, rows_v, sem):
        pltpu.sync_copy(idx_hbm.at[pl.ds(rb(g), GR)], idx_v)
        pltpu.async_copy(table_hbm.at[idx_v], rows_v, sem)

    fire(0, idx_v0, rows_v0, sem0)

    def pair(h, carry):
        g0 = 2 * h
        g1 = g0 + 1
        fire(g1, idx_v1, rows_v1, sem1)
        pltpu.make_async_copy(table_hbm.at[idx_v0], rows_v0, sem0).wait()
        pltpu.sync_copy(rows_v0, g_hbm.at[pl.ds(rb(g0), GR)])
        g2 = jnp.minimum(g0 + 2, NG - 1)
        fire(g2, idx_v0, rows_v0, sem0)
        pltpu.make_async_copy(table_hbm.at[idx_v1], rows_v1, sem1).wait()
        pltpu.sync_copy(rows_v1, g_hbm.at[pl.ds(rb(g1), GR)])
        return carry

    lax.fori_loop(0, NG // 2, pair, 0)
    pltpu.make_async_copy(table_hbm.at[idx_v0], rows_v0, sem0).wait()


def _sc_gather(table, idx_flat):
    M, Cp = table.shape
    R = idx_flat.shape[0]
    PPW = (R // KNN) // NW
    GP = 16
    NG = PPW // GP
    mesh = plsc.VectorSubcoreMesh(core_axis_name="c", subcore_axis_name="s")
    kcall = pl.kernel(
        functools.partial(_sc_gather_body, PPW=PPW, GP=GP, NG=NG),
        out_type=jax.ShapeDtypeStruct((R, Cp), jnp.float32),
        mesh=mesh,
        scratch_types=[
            pltpu.VMEM((GP * KNN,), jnp.int32),
            pltpu.VMEM((GP * KNN,), jnp.int32),
            pltpu.VMEM((GP * KNN, Cp), jnp.float32),
            pltpu.VMEM((GP * KNN, Cp), jnp.float32),
            pltpu.SemaphoreType.DMA,
            pltpu.SemaphoreType.DMA,
        ],
    )
    return kcall(table, idx_flat)


def _edge_kernel(g_ref, xb_ref, wt_ref, m_ref, ps_ref, ps2_ref):
    C2, Op = wt_ref.shape
    C = C2 // 2
    G = g_ref[...]
    xr = xb_ref[0]
    Gc = G[:, :C]
    xrc = xr[:, :C]
    xrep = jnp.broadcast_to(
        xrc[:, None, :], (xrc.shape[0], KNN, C)).reshape(xrc.shape[0] * KNN, C)
    f = jnp.concatenate([Gc - xrep, xrep], axis=1)
    y = lax.dot_general(
        f, wt_ref[...], (((1,), (0,)), ((), ())),
        preferred_element_type=jnp.float32)
    m_ref[0] = jnp.max(y.reshape(xrc.shape[0], KNN, Op), axis=1)
    ps_ref[0] = jnp.sum(y, axis=0, keepdims=True)
    ps2_ref[0] = jnp.sum(y * y, axis=0, keepdims=True)


def _layer_edge(G, xTp, WT):
    B, N, Cp = xTp.shape
    NB = N // BLK
    C2, Op = WT.shape
    nblk = B * NB
    return pl.pallas_call(
        _edge_kernel,
        grid=(B, NB),
        in_specs=[
            pl.BlockSpec((BLK * KNN, Cp), lambda b, i: (b * (N // BLK) + i, 0)),
            pl.BlockSpec((1, BLK, Cp), lambda b, i: (b, i, 0)),
            pl.BlockSpec((C2, Op), lambda b, i: (0, 0)),
        ],
        out_specs=[
            pl.BlockSpec((1, BLK, Op), lambda b, i: (b, i, 0)),
            pl.BlockSpec((1, 1, Op), lambda b, i: (b * (N // BLK) + i, 0, 0)),
            pl.BlockSpec((1, 1, Op), lambda b, i: (b * (N // BLK) + i, 0, 0)),
        ],
        out_shape=[
            jax.ShapeDtypeStruct((B, N, Op), jnp.float32),
            jax.ShapeDtypeStruct((nblk, 1, Op), jnp.float32),
            jax.ShapeDtypeStruct((nblk, 1, Op), jnp.float32),
        ],
    )(G, xTp, WT)


def _combine_kernel(m_ref, ps_ref, ps2_ref, g_ref, beta_ref, o_ref, *, cnt):
    mean = jnp.sum(ps_ref[...], axis=0, keepdims=True) / cnt
    ey2 = jnp.sum(ps2_ref[...], axis=0, keepdims=True) / cnt
    var = ey2 - mean * mean
    y = (m_ref[...] - mean) / jnp.sqrt(var + 1e-5) * g_ref[...] + beta_ref[...]
    o_ref[...] = jnp.where(y > 0, y, 0.2 * y)


def _combine(M2, ps, ps2, g, beta, cnt):
    Mrows, Op = M2.shape
    return pl.pallas_call(
        functools.partial(_combine_kernel, cnt=cnt),
        out_shape=jax.ShapeDtypeStruct((Mrows, Op), jnp.float32),
    )(M2, ps, ps2, g.reshape(1, Op), beta.reshape(1, Op))


_HBLK = 1024


def _mm_bias_kernel(x_ref, w_ref, b_ref, o_ref):
    o_ref[...] = b_ref[...] + jnp.dot(x_ref[...], w_ref[...],
                                      preferred_element_type=jnp.float32)


def _mm_bias(x, w, bias):
    M, _ = x.shape
    O = w.shape[1]
    return pl.pallas_call(
        _mm_bias_kernel,
        grid=(M // _HBLK,),
        in_specs=[pl.BlockSpec((_HBLK, x.shape[1]), lambda i: (i, 0)),
                  pl.BlockSpec(w.shape, lambda i: (0, 0)),
                  pl.BlockSpec((1, O), lambda i: (0, 0))],
        out_specs=pl.BlockSpec((_HBLK, O), lambda i: (i, 0)),
        out_shape=jax.ShapeDtypeStruct((M, O), jnp.float32),
    )(x, w, bias.reshape(1, O))


def _bn_relu_kernel(h_ref, g_ref, beta_ref, o_ref):
    h = h_ref[...]
    mean = jnp.mean(h, axis=0, keepdims=True)
    var = jnp.mean((h - mean) ** 2, axis=0, keepdims=True)
    o_ref[...] = jnp.maximum(
        (h - mean) / jnp.sqrt(var + 1e-5) * g_ref[...] + beta_ref[...], 0.0)


def _bn_relu(h, g, beta):
    M, O = h.shape
    return pl.pallas_call(
        _bn_relu_kernel,
        out_shape=jax.ShapeDtypeStruct((M, O), jnp.float32),
    )(h, g.reshape(1, O), beta.reshape(1, O))


def _edge_layer(xT_real, W, g, beta):
    B, N, C = xT_real.shape
    O = W.shape[0]
    Op = -(-O // 128) * 128
    Cp = -(-C // 128) * 128
    idx = _layer_knn(xT_real)
    xTp = xT_real if Cp == C else jnp.pad(xT_real, ((0, 0), (0, 0), (0, Cp - C)))
    G = _sc_gather(xTp.reshape(B * N, Cp), idx.reshape(-1))
    WT = jnp.zeros((2 * C, Op), jnp.float32).at[:, :O].set(W.T)
    gp = jnp.zeros((Op,), jnp.float32).at[:O].set(g)
    betap = jnp.zeros((Op,), jnp.float32).at[:O].set(beta)
    M2, ps, ps2 = _layer_edge(G, xTp, WT)
    out = _combine(M2.reshape(B * N, Op), ps.reshape(-1, Op),
                   ps2.reshape(-1, Op), gp, betap, float(B * N * KNN))
    return out.reshape(B, N, Op)


def kernel(x, W1, g1, b1, W2, g2, b2, W3, g3, b3, W4, g4, b4,
           Wc1, bc1, gc1, bnc1, Wc2, bc2, gc2, bnc2, Wc3, bc3):
    B, N, _ = x.shape
    sizes = (64, 64, 128, 256)
    xT = x
    reals = []
    for (W, g, beta), O in zip(((W1, g1, b1), (W2, g2, b2),
                                (W3, g3, b3), (W4, g4, b4)), sizes):
        xTp = _edge_layer(xT, W, g, beta)
        xT = xTp[..., :O]
        reals.append(xT.reshape(B * N, O))
    xcat = jnp.concatenate(reals, axis=1)
    h = _mm_bias(xcat, Wc1.T, bc1)
    h = _bn_relu(h, gc1, bnc1)
    h = _mm_bias(h, Wc2.T, bc2)
    h = _bn_relu(h, gc2, bnc2)
    out = _mm_bias(h, Wc3.T, bc3)
    return out.reshape(B, N)
